# baseline (speedup 1.0000x reference)
import functools

import jax
import jax.numpy as jnp
from jax import lax
from jax.experimental import pallas as pl
from jax.experimental.pallas import tpu as pltpu
from jax.experimental.pallas import tpu_sc as plsc

_DIM = 64
_TOT = 16384 * 26
_NW = 32
_PER_W = _TOT // _NW
_IR = 128
_NIR = _PER_W // _IR
_CH = 256
_NCH = _PER_W // _CH
_NSLOT = 4

_mesh = plsc.VectorSubcoreMesh(core_axis_name="c", subcore_axis_name="s")


def _make_gather():
    @functools.partial(
        pl.kernel,
        mesh=_mesh,
        out_type=jax.ShapeDtypeStruct((_TOT, _DIM), jnp.float32),
        compiler_params=pltpu.CompilerParams(use_tc_tiling_on_sc=False),
        scratch_types=[
            pltpu.VMEM((_NIR, _IR), jnp.int32),
            [pltpu.VMEM((_CH, _DIM), jnp.float32)] * _NSLOT,
            [pltpu.SemaphoreType.DMA] * _NSLOT,
            [pltpu.SemaphoreType.DMA] * _NSLOT,
        ],
    )
    def gather_kernel(idx_hbm, table_hbm, out_hbm, idx_v, rows, gsem, ssem):
        wid = lax.axis_index("s") * 2 + lax.axis_index("c")
        pltpu.sync_copy(idx_hbm.at[pl.ds(wid * _NIR, _NIR)], idx_v)
        base = wid * _PER_W

        def gfire(ci, s):
            return [
                pltpu.async_copy(
                    table_hbm.at[idx_v.at[ci * (_CH // _IR) + k]],
                    rows[s].at[pl.ds(k * _IR, _IR)],
                    gsem[s],
                )
                for k in range(_CH // _IR)
            ]

        def sfire(ci, s):
            return pltpu.async_copy(
                rows[s], out_hbm.at[pl.ds(base + ci * _CH, _CH)], ssem[s]
            )

        hg = [gfire(s, s) for s in range(_NSLOT)]
        hs = [None] * _NSLOT
        for h in hg[0]:
            h.wait()
        hs[0] = sfire(0, 0)
        for i in range(_NSLOT, _NCH):
            b = i % _NSLOT
            hs[b].wait()
            hg[b] = gfire(i, b)
            b2 = (b + 1) % _NSLOT
            for h in hg[b2]:
                h.wait()
            hs[b2] = sfire(i - (_NSLOT - 1), b2)
        for e in range(_NSLOT - 1):
            b2 = (_NCH + e + 1) % _NSLOT
            for h in hg[b2]:
                h.wait()
            hs[b2] = sfire(_NCH + e - (_NSLOT - 1), b2)
        for s in range(_NSLOT):
            hs[s].wait()

    return gather_kernel


_GATHER = _make_gather()


def kernel(indices, weight):
    flat = indices.reshape(_TOT // _IR, _IR)
    out = _GATHER(flat, weight)
    return out.reshape(indices.shape + (weight.shape[-1],))
